# baseline (speedup 1.0000x reference)
import functools

import jax
import jax.numpy as jnp
from jax import lax
from jax.experimental import pallas as pl
from jax.experimental.pallas import tpu as pltpu
from jax.experimental.pallas import tpu_sc as plsc

V = 8
N = 16384
D = 32
K = 512

N_BLK = 2048
NB = N // N_BLK
VH = V // 4

NW = 32
ROWS_PER_W = (VH * N) // NW
CH = 128
NCH = ROWS_PER_W // CH


CH_K = 8


def _make_dist_argmin(base_v):
    def _dist_argmin_kernel(xt_ref, wtn_ref, w2_ref, idx_ref, part_ref):
        v = pl.program_id(0)
        nb = pl.program_id(1)
        xt = xt_ref[0]
        wtn = wtn_ref[0]
        w2 = w2_ref[0]
        mmt = jnp.dot(wtn, xt, preferred_element_type=jnp.float32)
        val = jnp.full((CH_K, N_BLK), jnp.inf, jnp.float32)
        ich = jnp.zeros((CH_K, N_BLK), jnp.int32)
        for i in range(K // CH_K):
            sl = slice(i * CH_K, (i + 1) * CH_K)
            d = mmt[sl, :] + w2[sl, :]
            c = d < val
            val = jnp.where(c, d, val)
            ich = jnp.where(c, i, ich)
        sub = jax.lax.broadcasted_iota(jnp.int32, (CH_K, N_BLK), 0)
        kidx = ich * CH_K + sub
        m = jnp.min(val, axis=0, keepdims=True)
        kbest = jnp.min(jnp.where(val == m, kidx, K), axis=0)
        idx_ref[...] = kbest + (v + base_v) * K
        part_ref[v, nb] = jnp.sum(m) + jnp.sum(xt * xt)

    return pl.pallas_call(
        _dist_argmin_kernel,
        grid=(VH, NB),
        in_specs=[
            pl.BlockSpec((1, D, N_BLK), lambda v, nb: (v, 0, nb)),
            pl.BlockSpec((1, K, D), lambda v, nb: (v, 0, 0)),
            pl.BlockSpec((1, K, 1), lambda v, nb: (v, 0, 0)),
        ],
        out_specs=[
            pl.BlockSpec((N_BLK,), lambda v, nb: (v * NB + nb,)),
            pl.BlockSpec((VH, NB), lambda v, nb: (0, 0), memory_space=pltpu.SMEM),
        ],
        out_shape=[
            jax.ShapeDtypeStruct((VH * N,), jnp.int32),
            jax.ShapeDtypeStruct((VH, NB), jnp.float32),
        ],
    )


_dist_argmin_h = [_make_dist_argmin(h * VH) for h in range(V // VH)]


@functools.partial(
    pl.kernel,
    out_type=jax.ShapeDtypeStruct((VH, N, D), jnp.float32),
    mesh=plsc.VectorSubcoreMesh(core_axis_name="c", subcore_axis_name="s"),
    scratch_types=[
        pltpu.VMEM((ROWS_PER_W,), jnp.int32),
        pltpu.VMEM((CH, D), jnp.float32),
        pltpu.VMEM((CH, D), jnp.float32),
        pltpu.SemaphoreType.DMA,
        pltpu.SemaphoreType.DMA,
        pltpu.SemaphoreType.DMA,
        pltpu.SemaphoreType.DMA,
    ],
    compiler_params=pltpu.CompilerParams(use_tc_tiling_on_sc=False),
)
def _sc_gather(table_hbm, idx_hbm, out_hbm, idx_v, rows0, rows1, g0, g1, o0, o1):
    wid = lax.axis_index("s") * 2 + lax.axis_index("c")
    base = wid * ROWS_PER_W
    pltpu.sync_copy(idx_hbm.at[pl.ds(base, ROWS_PER_W)], idx_v)

    rows = (rows0, rows1)
    gsem = (g0, g1)
    osem = (o0, o1)
    g_prev = pltpu.async_copy(table_hbm.at[idx_v.at[pl.ds(0, CH)]], rows[0], gsem[0])
    o_pend = [None, None]
    for i in range(NCH):
        b = i & 1
        nb_ = 1 - b
        if i + 1 < NCH:
            if o_pend[nb_] is not None:
                o_pend[nb_].wait()
                o_pend[nb_] = None
            g_next = pltpu.async_copy(
                table_hbm.at[idx_v.at[pl.ds((i + 1) * CH, CH)]], rows[nb_], gsem[nb_])
        g_prev.wait()
        o_pend[b] = pltpu.async_copy(
            rows[b],
            out_hbm.at[wid // (N // ROWS_PER_W),
                       pl.ds((wid % (N // ROWS_PER_W)) * ROWS_PER_W + i * CH, CH)],
            osem[b])
        if i + 1 < NCH:
            g_prev = g_next
    for p in o_pend:
        if p is not None:
            p.wait()


def kernel(inputs, embeddings):
    xt = jnp.transpose(inputs, (0, 2, 1))
    wt = jnp.transpose(embeddings, (0, 2, 1))
    w2 = jnp.sum(embeddings ** 2, axis=1)[:, :, None]
    wtn = -2.0 * wt
    table = wt.reshape(V * K, D)
    outs, parts = [], []
    for h in range(V // VH):
        sl = slice(h * VH, (h + 1) * VH)
        idx_h, parts_h = _dist_argmin_h[h](xt[sl], wtn[sl], w2[sl])
        outs.append(_sc_gather(table, idx_h))
        parts.append(jnp.sum(parts_h))
    output = jnp.concatenate(outs, axis=0)
    loss = 1.25 * (sum(parts) / float(V * N * D))
    return output, loss


# speedup vs baseline: 1.2398x; 1.2398x over previous
import functools

import jax
import jax.numpy as jnp
from jax import lax
from jax.experimental import pallas as pl
from jax.experimental.pallas import tpu as pltpu
from jax.experimental.pallas import tpu_sc as plsc

V = 8
N = 16384
D = 32
K = 512

N_BLK = 2048
NB = N // N_BLK
VH = V // 2

NW = 32
ROWS_PER_W = (VH * N) // NW
CH = 128
NCH = ROWS_PER_W // CH


CH_K = 8


def _make_dist_argmin(base_v):
    def _dist_argmin_kernel(xt_ref, wtn_ref, idx_ref, part_ref):
        v = pl.program_id(0)
        nb = pl.program_id(1)
        xt = xt_ref[0]
        wtn = wtn_ref[0]
        mmt = jnp.dot(wtn, xt, preferred_element_type=jnp.float32)
        val = jnp.full((CH_K, N_BLK), jnp.inf, jnp.float32)
        ich = jnp.zeros((CH_K, N_BLK), jnp.int32)
        for i in range(K // CH_K):
            sl = slice(i * CH_K, (i + 1) * CH_K)
            d = mmt[sl, :]
            c = d < val
            val = jnp.where(c, d, val)
            ich = jnp.where(c, i, ich)
        sub = jax.lax.broadcasted_iota(jnp.int32, (CH_K, N_BLK), 0)
        kidx = ich * CH_K + sub
        m = jnp.min(val, axis=0, keepdims=True)
        kbest = jnp.min(jnp.where(val == m, kidx, K), axis=0)
        idx_ref[...] = kbest + (v + base_v) * K
        part_ref[v, nb] = jnp.sum(m) + (jnp.sum(xt * xt) - float(N_BLK))

    return pl.pallas_call(
        _dist_argmin_kernel,
        grid=(VH, NB),
        in_specs=[
            pl.BlockSpec((1, D + 1, N_BLK), lambda v, nb: (v, 0, nb)),
            pl.BlockSpec((1, K, D + 1), lambda v, nb: (v, 0, 0)),
        ],
        out_specs=[
            pl.BlockSpec((N_BLK,), lambda v, nb: (v * NB + nb,)),
            pl.BlockSpec((VH, NB), lambda v, nb: (0, 0), memory_space=pltpu.SMEM),
        ],
        out_shape=[
            jax.ShapeDtypeStruct((VH * N,), jnp.int32),
            jax.ShapeDtypeStruct((VH, NB), jnp.float32),
        ],
    )


_dist_argmin_h = [_make_dist_argmin(h * VH) for h in range(V // VH)]


@functools.partial(
    pl.kernel,
    out_type=jax.ShapeDtypeStruct((VH, N, D), jnp.float32),
    mesh=plsc.VectorSubcoreMesh(core_axis_name="c", subcore_axis_name="s"),
    scratch_types=[
        pltpu.VMEM((ROWS_PER_W,), jnp.int32),
        pltpu.VMEM((CH, D), jnp.float32),
        pltpu.VMEM((CH, D), jnp.float32),
        pltpu.SemaphoreType.DMA,
        pltpu.SemaphoreType.DMA,
        pltpu.SemaphoreType.DMA,
        pltpu.SemaphoreType.DMA,
    ],
    compiler_params=pltpu.CompilerParams(use_tc_tiling_on_sc=False),
)
def _sc_gather(table_hbm, idx_hbm, out_hbm, idx_v, rows0, rows1, g0, g1, o0, o1):
    wid = lax.axis_index("s") * 2 + lax.axis_index("c")
    base = wid * ROWS_PER_W
    pltpu.sync_copy(idx_hbm.at[pl.ds(base, ROWS_PER_W)], idx_v)

    rows = (rows0, rows1)
    gsem = (g0, g1)
    osem = (o0, o1)
    g_prev = pltpu.async_copy(table_hbm.at[idx_v.at[pl.ds(0, CH)]], rows[0], gsem[0])
    o_pend = [None, None]
    for i in range(NCH):
        b = i & 1
        nb_ = 1 - b
        if i + 1 < NCH:
            if o_pend[nb_] is not None:
                o_pend[nb_].wait()
                o_pend[nb_] = None
            g_next = pltpu.async_copy(
                table_hbm.at[idx_v.at[pl.ds((i + 1) * CH, CH)]], rows[nb_], gsem[nb_])
        g_prev.wait()
        o_pend[b] = pltpu.async_copy(
            rows[b],
            out_hbm.at[wid // (N // ROWS_PER_W),
                       pl.ds((wid % (N // ROWS_PER_W)) * ROWS_PER_W + i * CH, CH)],
            osem[b])
        if i + 1 < NCH:
            g_prev = g_next
    for p in o_pend:
        if p is not None:
            p.wait()


def kernel(inputs, embeddings):
    xt = jnp.concatenate(
        [jnp.transpose(inputs, (0, 2, 1)),
         jnp.ones((V, 1, N), jnp.float32)], axis=1)
    wt = jnp.transpose(embeddings, (0, 2, 1))
    w2 = jnp.sum(embeddings ** 2, axis=1)[:, :, None]
    wtn = jnp.concatenate([-2.0 * wt, w2], axis=2)
    table = wt.reshape(V * K, D)
    outs, parts = [], []
    for h in range(V // VH):
        sl = slice(h * VH, (h + 1) * VH)
        idx_h, parts_h = _dist_argmin_h[h](xt[sl], wtn[sl])
        outs.append(_sc_gather(table, idx_h))
        parts.append(jnp.sum(parts_h))
    output = jnp.concatenate(outs, axis=0)
    loss = 1.25 * (sum(parts) / float(V * N * D))
    return output, loss
